# baseline (speedup 1.0000x reference)
import functools
import math

import jax
import jax.numpy as jnp
from jax import lax
from jax.experimental import pallas as pl
from jax.experimental.pallas import tpu as pltpu
from jax.experimental.pallas import tpu_sc as plsc

B = 1024
L = 200
D = 128
SCALE = math.sqrt(float(D))

NC = 2
NS = 16
NW = NC * NS
HALF = L // 2
SPW = B // NW
NBUF = 3
LANES = 16
VECS_PER_ROW = D // LANES

_mesh = plsc.VectorSubcoreMesh(core_axis_name="c", subcore_axis_name="s")


@functools.partial(
    pl.kernel,
    out_type=jax.ShapeDtypeStruct((B * L, D), jnp.float32),
    mesh=_mesh,
    scratch_types=[
        pltpu.VMEM((2 * SPW, HALF), jnp.int32),
        [pltpu.VMEM((L, D), jnp.float32) for _ in range(NBUF)],
        pltpu.VMEM((L, D), jnp.float32),
        [[pltpu.SemaphoreType.DMA for _ in range(2)]
         for _ in range(NBUF)],
        [pltpu.SemaphoreType.DMA for _ in range(NBUF)],
        pltpu.SemaphoreType.DMA,
    ],
)
def _emb_kernel(x_hbm, w_hbm, pe_hbm, out_hbm, idx_v, rows, pe_v, gsem, wsem,
                psem):
    wid = lax.axis_index("s") * NC + lax.axis_index("c")
    base = wid * SPW

    pltpu.sync_copy(x_hbm.at[pl.ds(base * 2, 2 * SPW)], idx_v)
    pe_cp = pltpu.async_copy(pe_hbm.at[pl.ds(0, L)], pe_v, psem)

    def fire(t, bt):
        pltpu.async_copy(
            w_hbm.at[idx_v.at[2 * t]], rows[bt].at[pl.ds(0, HALF)],
            gsem[bt][0])
        pltpu.async_copy(
            w_hbm.at[idx_v.at[2 * t + 1]], rows[bt].at[pl.ds(HALF, HALF)],
            gsem[bt][1])

    def drain_gather_half(b, h):
        pltpu.make_async_copy(
            w_hbm.at[idx_v.at[0]], rows[b].at[pl.ds(h * HALF, HALF)],
            gsem[b][h]).wait()

    def drain_wb(b):
        pltpu.make_async_copy(
            rows[b], out_hbm.at[pl.ds(0, L)], wsem[b]).wait()

    def compute_half(b, h):
        def row_body(r, carry):
            for c in range(VECS_PER_ROW):
                sl = pl.ds(c * LANES, LANES)
                rows[b][r, sl] = rows[b][r, sl] * SCALE + pe_v[r, sl]
            return carry

        lax.fori_loop(h * HALF, (h + 1) * HALF, row_body, 0)

    def step(s, b, do_drain_wb, do_fire):
        drain_gather_half(b, 0)
        compute_half(b, 0)
        if do_drain_wb:
            drain_wb((b + 2) % NBUF)
        if do_fire:
            fire(s + 2, (b + 2) % NBUF)
        drain_gather_half(b, 1)
        compute_half(b, 1)
        pltpu.async_copy(
            rows[b], out_hbm.at[pl.ds((base + s) * L, L)], wsem[b])

    fire(0, 0)
    fire(1, 1)
    pe_cp.wait()

    step(0, 0, False, True)
    step(1, 1, True, True)
    step(2, 2, True, True)

    def group_body(g, carry):
        for b in range(NBUF):
            step(NBUF * g + b, b, True, True)
        return carry

    lax.fori_loop(1, SPW // NBUF, group_body, 0)

    step(SPW - 2, 0, False, False)
    step(SPW - 1, 1, False, False)

    for b in range(NBUF):
        drain_wb(b)


def kernel(x, W, pe):
    x2 = x.reshape(B * L // HALF, HALF)
    out = _emb_kernel(x2, W, pe)
    return out.reshape(B, L, D)


# speedup vs baseline: 1.0036x; 1.0036x over previous
import functools
import math

import jax
import jax.numpy as jnp
from jax import lax
from jax.experimental import pallas as pl
from jax.experimental.pallas import tpu as pltpu
from jax.experimental.pallas import tpu_sc as plsc

B = 1024
L = 200
D = 128
SCALE = math.sqrt(float(D))

NC = 2
NS = 16
NW = NC * NS
HALF = L // 2
SPW = B // NW
NBUF = 3
LANES = 16
VECS_PER_ROW = D // LANES

_mesh = plsc.VectorSubcoreMesh(core_axis_name="c", subcore_axis_name="s")


@functools.partial(
    pl.kernel,
    out_type=jax.ShapeDtypeStruct((B * L, D), jnp.float32),
    mesh=_mesh,
    scratch_types=[
        pltpu.VMEM((2 * SPW, HALF), jnp.int32),
        [pltpu.VMEM((L, D), jnp.float32) for _ in range(NBUF)],
        pltpu.VMEM((L, D), jnp.float32),
        [[pltpu.SemaphoreType.DMA for _ in range(2)]
         for _ in range(NBUF)],
        [pltpu.SemaphoreType.DMA for _ in range(NBUF)],
        pltpu.SemaphoreType.DMA,
    ],
)
def _emb_kernel(x_hbm, w_hbm, pe_hbm, out_hbm, idx_v, rows, pe_v, gsem, wsem,
                psem):
    wid = lax.axis_index("s") * NC + lax.axis_index("c")
    base = wid * SPW

    pltpu.sync_copy(x_hbm.at[pl.ds(base * 2, 2 * SPW)], idx_v)
    pe_cp = pltpu.async_copy(pe_hbm.at[pl.ds(0, L)], pe_v, psem)

    def fire(t, bt):
        pltpu.async_copy(
            w_hbm.at[idx_v.at[2 * t]], rows[bt].at[pl.ds(0, HALF)],
            gsem[bt][0])
        pltpu.async_copy(
            w_hbm.at[idx_v.at[2 * t + 1]], rows[bt].at[pl.ds(HALF, HALF)],
            gsem[bt][1])

    def drain_gather_half(b, h):
        pltpu.make_async_copy(
            w_hbm.at[idx_v.at[0]], rows[b].at[pl.ds(h * HALF, HALF)],
            gsem[b][h]).wait()

    def drain_wb(b):
        pltpu.make_async_copy(
            rows[b], out_hbm.at[pl.ds(0, L)], wsem[b]).wait()

    WB_SPLIT = 96

    def compute_rows(b, lo, hi):
        def row_body(r, carry):
            for c in range(VECS_PER_ROW):
                sl = pl.ds(c * LANES, LANES)
                rows[b][r, sl] = rows[b][r, sl] * SCALE + pe_v[r, sl]
            return carry

        lax.fori_loop(lo, hi, row_body, 0)

    def step(s, b, do_drain_wb, do_fire):
        drain_gather_half(b, 0)
        compute_rows(b, 0, WB_SPLIT)
        pltpu.async_copy(
            rows[b].at[pl.ds(0, WB_SPLIT)],
            out_hbm.at[pl.ds((base + s) * L, WB_SPLIT)], wsem[b])
        if do_drain_wb:
            drain_wb((b + 2) % NBUF)
        if do_fire:
            fire(s + 2, (b + 2) % NBUF)
        drain_gather_half(b, 1)
        compute_rows(b, WB_SPLIT, L)
        pltpu.async_copy(
            rows[b].at[pl.ds(WB_SPLIT, L - WB_SPLIT)],
            out_hbm.at[pl.ds((base + s) * L + WB_SPLIT, L - WB_SPLIT)],
            wsem[b])

    fire(0, 0)
    fire(1, 1)
    pe_cp.wait()

    step(0, 0, False, True)
    step(1, 1, True, True)
    step(2, 2, True, True)

    def group_body(g, carry):
        for b in range(NBUF):
            step(NBUF * g + b, b, True, True)
        return carry

    lax.fori_loop(1, SPW // NBUF, group_body, 0)

    step(SPW - 2, 0, False, False)
    step(SPW - 1, 1, False, False)

    for b in range(NBUF):
        drain_wb(b)


def kernel(x, W, pe):
    x2 = x.reshape(B * L // HALF, HALF)
    out = _emb_kernel(x2, W, pe)
    return out.reshape(B, L, D)
